# Initial kernel scaffold; baseline (speedup 1.0000x reference)
import jax
import jax.numpy as jnp
from jax.experimental import pallas as pl


def kernel(x, edge_index, edge_attr, batch, lin1_W, lin1_b, lin2_W, lin2_b, conv_b, gru_Wih, gru_Whh, gru_bih, gru_bhh, lstm_Wih, lstm_Whh, lstm_bih, lstm_bhh, fc1_W, fc1_b, fc2_W, fc2_b):
    raise NotImplementedError("write your pallas kernel here")


# speedup vs baseline: 2.4101x; 2.4101x over previous
import functools

import jax
import jax.numpy as jnp
from jax import lax
from jax.experimental import pallas as pl
from jax.experimental.pallas import tpu as pltpu
from jax.experimental.pallas import tpu_sc as plsc

DIM = 73
N = 10000
E = 80000
G = 64

XP = 80
KH = 128
ZW = XP * KH
E_PAD = 81920
E_B = 256
HG_B = 2048

NC = 2
NS = 16
NW = NC * NS
CH = 128
EPW = E_PAD // NW
EPC = E_PAD // NC // NS
ROWS_PER_TILE = N // NS


def _hgate_body(ea_ref, w_ref, b_ref, out_ref):
    h = jnp.dot(ea_ref[...], w_ref[...], preferred_element_type=jnp.float32)
    out_ref[...] = jnp.maximum(h + b_ref[...], 0.0).astype(jnp.bfloat16)


def _msg_body(xs_ref, hg_ref, b_ref, out_ref):
    xs = xs_ref[...].astype(jnp.bfloat16)
    hg = hg_ref[...]
    z = (xs[:, :, None] * hg[:, None, :]).reshape(E_B, ZW)
    out_ref[...] = jnp.dot(z, b_ref[...], preferred_element_type=jnp.float32)


def _gru_body(p_ref, h_ref, wi_ref, wh_ref, bi_ref, bh_ref, cb_ref,
              mask_ref, out_ref):
    m = jnp.maximum(p_ref[0] + p_ref[1] + cb_ref[...], 0.0)
    h = h_ref[...]
    def mm(a, w):
        return jnp.dot(a, w, preferred_element_type=jnp.float32)
    i_r = mm(m, wi_ref[0]) + bi_ref[0]
    i_z = mm(m, wi_ref[1]) + bi_ref[1]
    i_n = mm(m, wi_ref[2]) + bi_ref[2]
    h_r = mm(h, wh_ref[0]) + bh_ref[0]
    h_z = mm(h, wh_ref[1]) + bh_ref[1]
    h_n = mm(h, wh_ref[2]) + bh_ref[2]
    r = jax.nn.sigmoid(i_r + h_r)
    z = jax.nn.sigmoid(i_z + h_z)
    n = jnp.tanh(i_n + r * h_n)
    out_ref[...] = ((1.0 - z) * n + z * h) * mask_ref[...]


def _s2s_body(out_ref, batch_ref, wih_ref, whh_ref, lb_ref, fc1_ref,
              fc1b_ref, fc2_ref, fc2b_ref, y_ref):
    outx = out_ref[...]
    onehot = batch_ref[...] == lax.broadcasted_iota(jnp.int32, (N, G), 1)
    def mm(a, w):
        return jnp.dot(a, w, preferred_element_type=jnp.float32)
    qh = jnp.zeros((G, XP), jnp.float32)
    qc = jnp.zeros((G, XP), jnp.float32)
    q_star = jnp.zeros((G, 2 * XP), jnp.float32)
    for _ in range(3):
        g_i = jax.nn.sigmoid(mm(q_star, wih_ref[0]) + mm(qh, whh_ref[0]) + lb_ref[0])
        g_f = jax.nn.sigmoid(mm(q_star, wih_ref[1]) + mm(qh, whh_ref[1]) + lb_ref[1])
        g_g = jnp.tanh(mm(q_star, wih_ref[2]) + mm(qh, whh_ref[2]) + lb_ref[2])
        g_o = jax.nn.sigmoid(mm(q_star, wih_ref[3]) + mm(qh, whh_ref[3]) + lb_ref[3])
        qc = g_f * qc + g_i * g_g
        qh = g_o * jnp.tanh(qc)
        s = mm(outx, qh.T)
        sm = jnp.where(onehot, s, -1e30)
        emax = jnp.max(sm, axis=0, keepdims=True)
        p = jnp.where(onehot, jnp.exp(s - emax), 0.0)
        denom = jnp.sum(p, axis=0, keepdims=True)
        a = p / jnp.maximum(denom, 1e-30)
        r_g = lax.dot_general(a, outx, (((0,), (0,)), ((), ())),
                              preferred_element_type=jnp.float32)
        q_star = jnp.concatenate([qh, r_g], axis=1)
    y = jnp.maximum(mm(q_star, fc1_ref[...]) + fc1b_ref[...], 0.0)
    y_ref[...] = mm(y, fc2_ref[...]) + fc2b_ref[...]


@functools.lru_cache(maxsize=None)
def _sc_kernels():
    mesh = plsc.VectorSubcoreMesh(core_axis_name="c", subcore_axis_name="s",
                                  num_cores=NC, num_subcores=NS)

    @functools.partial(
        pl.kernel,
        out_type=jax.ShapeDtypeStruct((E_PAD, XP), jnp.float32),
        scratch_types=[
            pltpu.VMEM((CH,), jnp.int32),
            pltpu.VMEM((CH, XP), jnp.float32),
            pltpu.SemaphoreType.DMA,
        ],
        mesh=mesh,
        compiler_params=pltpu.CompilerParams(use_tc_tiling_on_sc=False),
    )
    def sc_gather(nodes_hbm, src_hbm, out_hbm, idx_v, rows_v, sem):
        wid = lax.axis_index("s") * NC + lax.axis_index("c")

        def body(j, carry):
            base = wid * EPW + j * CH
            pltpu.sync_copy(src_hbm.at[pl.ds(base, CH)], idx_v)
            pltpu.async_copy(nodes_hbm.at[idx_v], rows_v, sem).wait()
            pltpu.sync_copy(rows_v, out_hbm.at[pl.ds(base, CH)])
            return carry

        lax.fori_loop(0, EPW // CH, body, 0)

    @functools.partial(
        pl.kernel,
        out_type=jax.ShapeDtypeStruct((NC, N, XP), jnp.float32),
        scratch_types=[
            pltpu.VMEM((CH,), jnp.int32),
            pltpu.VMEM((CH, XP), jnp.float32),
            pltpu.VMEM_SHARED((N, XP), jnp.float32),
        ],
        mesh=mesh,
        compiler_params=pltpu.CompilerParams(use_tc_tiling_on_sc=False),
    )
    def sc_scatter(msg_hbm, dst_hbm, zero_hbm, out_hbm, idx_v, rows_v, acc_sh):
        c = lax.axis_index("c")
        s = lax.axis_index("s")
        pltpu.sync_copy(zero_hbm.at[pl.ds(s * ROWS_PER_TILE, ROWS_PER_TILE)],
                        acc_sh.at[pl.ds(s * ROWS_PER_TILE, ROWS_PER_TILE)])
        plsc.subcore_barrier()

        def body(j, carry):
            base = c * (E_PAD // NC) + s * EPC + j * CH
            pltpu.sync_copy(dst_hbm.at[pl.ds(base, CH)], idx_v)
            pltpu.sync_copy(msg_hbm.at[pl.ds(base, CH)], rows_v)
            pltpu.sync_copy(rows_v, acc_sh.at[idx_v], add=True)
            return carry

        lax.fori_loop(0, EPC // CH, body, 0)
        plsc.subcore_barrier()
        pltpu.sync_copy(acc_sh.at[pl.ds(s * ROWS_PER_TILE, ROWS_PER_TILE)],
                        out_hbm.at[c, pl.ds(s * ROWS_PER_TILE, ROWS_PER_TILE)])

    return sc_gather, sc_scatter


def _pad2(a, r, c):
    return jnp.pad(a, ((0, r - a.shape[0]), (0, c - a.shape[1])))


def kernel(x, edge_index, edge_attr, batch, lin1_W, lin1_b, lin2_W, lin2_b,
           conv_b, gru_Wih, gru_Whh, gru_bih, gru_bhh, lstm_Wih, lstm_Whh,
           lstm_bih, lstm_bhh, fc1_W, fc1_b, fc2_W, fc2_b):
    f32 = jnp.float32
    x_pad = _pad2(x, N, XP)
    src_p = jnp.pad(edge_index[0], (0, E_PAD - E))
    dst_p = jnp.pad(edge_index[1], (0, E_PAD - E))
    ea_p = _pad2(edge_attr, E_PAD, 8)
    lin1T = _pad2(lin1_W.T, 8, KH)
    lin1b2 = lin1_b.reshape(1, KH)

    l2r = lin2_W.reshape(DIM, DIM, KH)
    bm = jnp.pad(jnp.transpose(l2r, (0, 2, 1)),
                 ((0, XP - DIM), (0, 0), (0, XP - DIM)))
    B_bf = bm.reshape(ZW, XP).astype(jnp.bfloat16)

    conv_b2 = jnp.pad(conv_b, (0, XP - DIM)).reshape(1, XP)
    colmask = (jnp.arange(XP) < DIM).astype(f32).reshape(1, XP)

    def pad_sq(w):
        return _pad2(w.T, XP, XP)

    wi = jnp.stack([pad_sq(gru_Wih[g * DIM:(g + 1) * DIM]) for g in range(3)])
    wh = jnp.stack([pad_sq(gru_Whh[g * DIM:(g + 1) * DIM]) for g in range(3)])
    bi = jnp.pad(gru_bih.reshape(3, 1, DIM), ((0, 0), (0, 0), (0, XP - DIM)))
    bh = jnp.pad(gru_bhh.reshape(3, 1, DIM), ((0, 0), (0, 0), (0, XP - DIM)))

    def pad_wih(wg):
        top = _pad2(wg[:, :DIM].T, XP, XP)
        bot = _pad2(wg[:, DIM:].T, XP, XP)
        return jnp.concatenate([top, bot], axis=0)

    lwih = jnp.stack([pad_wih(lstm_Wih[g * DIM:(g + 1) * DIM]) for g in range(4)])
    lwhh = jnp.stack([pad_sq(lstm_Whh[g * DIM:(g + 1) * DIM]) for g in range(4)])
    lb = jnp.pad((lstm_bih + lstm_bhh).reshape(4, 1, DIM),
                 ((0, 0), (0, 0), (0, XP - DIM)))

    fc1T = jnp.zeros((2 * XP, XP), f32)
    fc1T = fc1T.at[:DIM, :DIM].set(fc1_W[:, :DIM].T)
    fc1T = fc1T.at[XP:XP + DIM, :DIM].set(fc1_W[:, DIM:].T)
    fc1b2 = jnp.pad(fc1_b, (0, XP - DIM)).reshape(1, XP)
    fc2T = _pad2(fc2_W.T, XP, KH)
    fc2b2 = jnp.pad(fc2_b, (0, KH - 1)).reshape(1, KH)
    batch2 = batch.reshape(N, 1)
    zeros_nxp = jnp.zeros((N, XP), f32)

    hg = pl.pallas_call(
        _hgate_body,
        grid=(E_PAD // HG_B,),
        in_specs=[
            pl.BlockSpec((HG_B, 8), lambda i: (i, 0)),
            pl.BlockSpec((8, KH), lambda i: (0, 0)),
            pl.BlockSpec((1, KH), lambda i: (0, 0)),
        ],
        out_specs=pl.BlockSpec((HG_B, KH), lambda i: (i, 0)),
        out_shape=jax.ShapeDtypeStruct((E_PAD, KH), jnp.bfloat16),
    )(ea_p, lin1T, lin1b2)

    msg_call = pl.pallas_call(
        _msg_body,
        grid=(E_PAD // E_B,),
        in_specs=[
            pl.BlockSpec((E_B, XP), lambda i: (i, 0)),
            pl.BlockSpec((E_B, KH), lambda i: (i, 0)),
            pl.BlockSpec((ZW, XP), lambda i: (0, 0)),
        ],
        out_specs=pl.BlockSpec((E_B, XP), lambda i: (i, 0)),
        out_shape=jax.ShapeDtypeStruct((E_PAD, XP), f32),
    )

    gru_call = pl.pallas_call(
        _gru_body,
        out_shape=jax.ShapeDtypeStruct((N, XP), f32),
    )

    sc_gather, sc_scatter = _sc_kernels()
    h = x_pad
    for _ in range(3):
        xs = sc_gather(h, src_p)
        msg = msg_call(xs, hg, B_bf)
        parts = sc_scatter(msg, dst_p, zeros_nxp)
        h = gru_call(parts, h, wi, wh, bi, bh, conv_b2, colmask)

    y = pl.pallas_call(
        _s2s_body,
        out_shape=jax.ShapeDtypeStruct((G, KH), f32),
    )(h, batch2, lwih, lwhh, lb, fc1T, fc1b2, fc2T, fc2b2)
    return y[:, 0]


# speedup vs baseline: 4.1693x; 1.7299x over previous
import functools

import jax
import jax.numpy as jnp
from jax import lax
from jax.experimental import pallas as pl
from jax.experimental.pallas import tpu as pltpu
from jax.experimental.pallas import tpu_sc as plsc

DIM = 73
N = 10000
E = 80000
G = 64

XP = 80
KH = 128
ZW = XP * KH
E_PAD = 81920
E_B = 256
HG_B = 2048

NC = 2
NS = 16
NW = NC * NS
CH = 128
EPW = E_PAD // NW
EPC = E_PAD // NC // NS
ROWS_PER_TILE = N // NS


def _hgate_body(eat_ref, w_ref, b_ref, out_ref):
    h = jnp.dot(w_ref[...], eat_ref[...], preferred_element_type=jnp.float32)
    out_ref[...] = jnp.maximum(h + b_ref[...], 0.0).astype(jnp.bfloat16)


def _msg_body(xs_ref, hgt_ref, bt_ref, out_ref):
    xst = jnp.transpose(xs_ref[...].astype(jnp.bfloat16))[:DIM]
    hgt = hgt_ref[...]
    zt = (xst[:, None, :] * hgt[None, :, :]).reshape(DIM * KH, E_B)
    mt = jnp.dot(bt_ref[...], zt, preferred_element_type=jnp.float32)
    out_ref[...] = jnp.transpose(mt)


def _gru_body(p_ref, h_ref, wi_ref, wh_ref, bi_ref, bh_ref, cb_ref,
              mask_ref, out_ref):
    m = jnp.maximum(p_ref[0] + p_ref[1] + cb_ref[...], 0.0)
    h = h_ref[...]
    def mm(a, w):
        return jnp.dot(a, w, preferred_element_type=jnp.float32)
    i_r = mm(m, wi_ref[0]) + bi_ref[0]
    i_z = mm(m, wi_ref[1]) + bi_ref[1]
    i_n = mm(m, wi_ref[2]) + bi_ref[2]
    h_r = mm(h, wh_ref[0]) + bh_ref[0]
    h_z = mm(h, wh_ref[1]) + bh_ref[1]
    h_n = mm(h, wh_ref[2]) + bh_ref[2]
    r = jax.nn.sigmoid(i_r + h_r)
    z = jax.nn.sigmoid(i_z + h_z)
    n = jnp.tanh(i_n + r * h_n)
    out_ref[...] = ((1.0 - z) * n + z * h) * mask_ref[...]


def _s2s_body(out_ref, batch_ref, wih_ref, whh_ref, lb_ref, fc1_ref,
              fc1b_ref, fc2_ref, fc2b_ref, y_ref):
    outx = out_ref[...]
    onehot = batch_ref[...] == lax.broadcasted_iota(jnp.int32, (N, G), 1)
    def mm(a, w):
        return jnp.dot(a, w, preferred_element_type=jnp.float32)
    qh = jnp.zeros((G, XP), jnp.float32)
    qc = jnp.zeros((G, XP), jnp.float32)
    q_star = jnp.zeros((G, 2 * XP), jnp.float32)
    for _ in range(3):
        g_i = jax.nn.sigmoid(mm(q_star, wih_ref[0]) + mm(qh, whh_ref[0]) + lb_ref[0])
        g_f = jax.nn.sigmoid(mm(q_star, wih_ref[1]) + mm(qh, whh_ref[1]) + lb_ref[1])
        g_g = jnp.tanh(mm(q_star, wih_ref[2]) + mm(qh, whh_ref[2]) + lb_ref[2])
        g_o = jax.nn.sigmoid(mm(q_star, wih_ref[3]) + mm(qh, whh_ref[3]) + lb_ref[3])
        qc = g_f * qc + g_i * g_g
        qh = g_o * jnp.tanh(qc)
        s = mm(outx, qh.T)
        sm = jnp.where(onehot, s, -1e30)
        emax = jnp.max(sm, axis=0, keepdims=True)
        p = jnp.where(onehot, jnp.exp(s - emax), 0.0)
        denom = jnp.sum(p, axis=0, keepdims=True)
        a = p / jnp.maximum(denom, 1e-30)
        r_g = lax.dot_general(a, outx, (((0,), (0,)), ((), ())),
                              preferred_element_type=jnp.float32)
        q_star = jnp.concatenate([qh, r_g], axis=1)
    y = jnp.maximum(mm(q_star, fc1_ref[...]) + fc1b_ref[...], 0.0)
    y_ref[...] = mm(y, fc2_ref[...]) + fc2b_ref[...]


@functools.lru_cache(maxsize=None)
def _sc_kernels():
    mesh = plsc.VectorSubcoreMesh(core_axis_name="c", subcore_axis_name="s",
                                  num_cores=NC, num_subcores=NS)

    n_chunks = EPW // CH
    mega = n_chunks // 2
    mrows = mega * CH

    @functools.partial(
        pl.kernel,
        out_type=jax.ShapeDtypeStruct((E_PAD, XP), jnp.float32),
        scratch_types=[
            pltpu.VMEM((n_chunks, CH), jnp.int32),
            pltpu.VMEM((mrows, XP), jnp.float32),
            pltpu.SemaphoreType.DMA,
        ],
        mesh=mesh,
        compiler_params=pltpu.CompilerParams(use_tc_tiling_on_sc=False),
    )
    def sc_gather(nodes_hbm, src2_hbm, out_hbm, idx_v, rows_v, sem):
        wid = lax.axis_index("s") * NC + lax.axis_index("c")
        pltpu.sync_copy(src2_hbm.at[pl.ds(wid * n_chunks, n_chunks)], idx_v)
        for m in range(2):
            handles = [
                pltpu.async_copy(nodes_hbm.at[idx_v.at[m * mega + j]],
                                 rows_v.at[pl.ds(j * CH, CH)], sem)
                for j in range(mega)
            ]
            for h in handles:
                h.wait()
            pltpu.sync_copy(rows_v,
                            out_hbm.at[pl.ds(wid * EPW + m * mrows, mrows)])

    mega_s = 5
    srows = mega_s * CH

    @functools.partial(
        pl.kernel,
        out_type=jax.ShapeDtypeStruct((NC, N, XP), jnp.float32),
        scratch_types=[
            pltpu.VMEM((n_chunks, CH), jnp.int32),
            pltpu.VMEM((srows, XP), jnp.float32),
            pltpu.VMEM_SHARED((N, XP), jnp.float32),
            pltpu.SemaphoreType.DMA,
        ],
        mesh=mesh,
        compiler_params=pltpu.CompilerParams(use_tc_tiling_on_sc=False),
    )
    def sc_scatter(msg_hbm, dst2_hbm, zero_hbm, out_hbm, idx_v, rows_v,
                   acc_sh, sem):
        c = lax.axis_index("c")
        s = lax.axis_index("s")
        pltpu.sync_copy(zero_hbm.at[pl.ds(s * ROWS_PER_TILE, ROWS_PER_TILE)],
                        acc_sh.at[pl.ds(s * ROWS_PER_TILE, ROWS_PER_TILE)])
        plsc.subcore_barrier()
        chunk0 = c * (E_PAD // NC // CH) + s * n_chunks
        pltpu.sync_copy(dst2_hbm.at[pl.ds(chunk0, n_chunks)], idx_v)
        for m in range(n_chunks // mega_s):
            pltpu.sync_copy(
                msg_hbm.at[pl.ds((chunk0 + m * mega_s) * CH, srows)], rows_v)
            handles = [
                pltpu.async_copy(rows_v.at[pl.ds(j * CH, CH)],
                                 acc_sh.at[idx_v.at[m * mega_s + j]], sem,
                                 add=True)
                for j in range(mega_s)
            ]
            for h in handles:
                h.wait()
        plsc.subcore_barrier()
        pltpu.sync_copy(acc_sh.at[pl.ds(s * ROWS_PER_TILE, ROWS_PER_TILE)],
                        out_hbm.at[c, pl.ds(s * ROWS_PER_TILE, ROWS_PER_TILE)])

    return sc_gather, sc_scatter


def _pad2(a, r, c):
    return jnp.pad(a, ((0, r - a.shape[0]), (0, c - a.shape[1])))


def kernel(x, edge_index, edge_attr, batch, lin1_W, lin1_b, lin2_W, lin2_b,
           conv_b, gru_Wih, gru_Whh, gru_bih, gru_bhh, lstm_Wih, lstm_Whh,
           lstm_bih, lstm_bhh, fc1_W, fc1_b, fc2_W, fc2_b):
    f32 = jnp.float32
    x_pad = _pad2(x, N, XP)
    src2 = jnp.pad(edge_index[0], (0, E_PAD - E)).reshape(E_PAD // CH, CH)
    dst2 = jnp.pad(edge_index[1], (0, E_PAD - E)).reshape(E_PAD // CH, CH)
    eaT = _pad2(edge_attr, E_PAD, 8).T
    lin1P = _pad2(lin1_W, KH, 8)
    lin1b2 = lin1_b.reshape(KH, 1)

    l2r = lin2_W.reshape(DIM, DIM, KH)
    BT_bf = _pad2(jnp.transpose(l2r, (1, 0, 2)).reshape(DIM, DIM * KH),
                  XP, DIM * KH).astype(jnp.bfloat16)

    conv_b2 = jnp.pad(conv_b, (0, XP - DIM)).reshape(1, XP)
    colmask = (jnp.arange(XP) < DIM).astype(f32).reshape(1, XP)

    def pad_sq(w):
        return _pad2(w.T, XP, XP)

    wi = jnp.stack([pad_sq(gru_Wih[g * DIM:(g + 1) * DIM]) for g in range(3)])
    wh = jnp.stack([pad_sq(gru_Whh[g * DIM:(g + 1) * DIM]) for g in range(3)])
    bi = jnp.pad(gru_bih.reshape(3, 1, DIM), ((0, 0), (0, 0), (0, XP - DIM)))
    bh = jnp.pad(gru_bhh.reshape(3, 1, DIM), ((0, 0), (0, 0), (0, XP - DIM)))

    def pad_wih(wg):
        top = _pad2(wg[:, :DIM].T, XP, XP)
        bot = _pad2(wg[:, DIM:].T, XP, XP)
        return jnp.concatenate([top, bot], axis=0)

    lwih = jnp.stack([pad_wih(lstm_Wih[g * DIM:(g + 1) * DIM]) for g in range(4)])
    lwhh = jnp.stack([pad_sq(lstm_Whh[g * DIM:(g + 1) * DIM]) for g in range(4)])
    lb = jnp.pad((lstm_bih + lstm_bhh).reshape(4, 1, DIM),
                 ((0, 0), (0, 0), (0, XP - DIM)))

    fc1T = jnp.zeros((2 * XP, XP), f32)
    fc1T = fc1T.at[:DIM, :DIM].set(fc1_W[:, :DIM].T)
    fc1T = fc1T.at[XP:XP + DIM, :DIM].set(fc1_W[:, DIM:].T)
    fc1b2 = jnp.pad(fc1_b, (0, XP - DIM)).reshape(1, XP)
    fc2T = _pad2(fc2_W.T, XP, KH)
    fc2b2 = jnp.pad(fc2_b, (0, KH - 1)).reshape(1, KH)
    batch2 = batch.reshape(N, 1)
    zeros_nxp = jnp.zeros((N, XP), f32)

    hgt = pl.pallas_call(
        _hgate_body,
        grid=(E_PAD // HG_B,),
        in_specs=[
            pl.BlockSpec((8, HG_B), lambda i: (0, i)),
            pl.BlockSpec((KH, 8), lambda i: (0, 0)),
            pl.BlockSpec((KH, 1), lambda i: (0, 0)),
        ],
        out_specs=pl.BlockSpec((KH, HG_B), lambda i: (0, i)),
        out_shape=jax.ShapeDtypeStruct((KH, E_PAD), jnp.bfloat16),
    )(eaT, lin1P, lin1b2)

    msg_call = pl.pallas_call(
        _msg_body,
        grid=(E_PAD // E_B,),
        in_specs=[
            pl.BlockSpec((E_B, XP), lambda i: (i, 0)),
            pl.BlockSpec((KH, E_B), lambda i: (0, i)),
            pl.BlockSpec((XP, DIM * KH), lambda i: (0, 0)),
        ],
        out_specs=pl.BlockSpec((E_B, XP), lambda i: (i, 0)),
        out_shape=jax.ShapeDtypeStruct((E_PAD, XP), f32),
    )

    gru_call = pl.pallas_call(
        _gru_body,
        out_shape=jax.ShapeDtypeStruct((N, XP), f32),
    )

    sc_gather, sc_scatter = _sc_kernels()
    h = x_pad
    for _ in range(3):
        xs = sc_gather(h, src2)
        msg = msg_call(xs, hgt, BT_bf)
        parts = sc_scatter(msg, dst2, zeros_nxp)
        h = gru_call(parts, h, wi, wh, bi, bh, conv_b2, colmask)

    y = pl.pallas_call(
        _s2s_body,
        out_shape=jax.ShapeDtypeStruct((G, KH), f32),
    )(h, batch2, lwih, lwhh, lb, fc1T, fc1b2, fc2T, fc2b2)
    return y[:, 0]
